# Initial kernel scaffold; baseline (speedup 1.0000x reference)
import jax
import jax.numpy as jnp
from jax.experimental import pallas as pl


def kernel(tod, dow, tod_table, dow_table):
    raise NotImplementedError("write your pallas kernel here")


# speedup vs baseline: 5.1414x; 5.1414x over previous
import functools

import jax
import jax.numpy as jnp
from jax import lax
from jax.experimental import pallas as pl
from jax.experimental.pallas import tpu as pltpu
from jax.experimental.pallas import tpu_sc as plsc

B, T = 16384, 200
STEPS_PER_DAY = 288
EMB = 16
OUT_D = 2 * EMB
N = B * T
NW = 32
RPW = N // NW
C = 1280
NCHUNK = RPW // C
GPC = C // 16
DOW_BASE = STEPS_PER_DAY * EMB
CTAB_LEN = DOW_BASE + 7 * EMB


def _body(tod_hbm, dow_hbm, ctab_hbm, out_hbm, ctab_v, tod_v, dow_v, out_v):
    wid = lax.axis_index("s") * 2 + lax.axis_index("c")
    base = wid * RPW
    pltpu.sync_copy(ctab_hbm, ctab_v)
    lane = lax.iota(jnp.int32, 16)

    def chunk_body(ci, _):
        cb = base + ci * C
        pltpu.sync_copy(tod_hbm.at[pl.ds(cb, C)], tod_v)
        pltpu.sync_copy(dow_hbm.at[pl.ds(cb, C)], dow_v)

        def group_body(g, _):
            tv = tod_v[pl.ds(g * 16, 16)]
            dv = dow_v[pl.ds(g * 16, 16)]
            ti = (tv * jnp.float32(STEPS_PER_DAY)).astype(jnp.int32)
            ti = jnp.minimum(jnp.maximum(ti, 0), STEPS_PER_DAY - 1)
            di = jnp.minimum(jnp.maximum(dv, 0), 6)
            tbase = ti * EMB
            dbase = di * EMB + DOW_BASE
            obase = lane * OUT_D + g * (16 * OUT_D)
            for d in range(EMB):
                gt = plsc.load_gather(ctab_v, [tbase + d])
                plsc.store_scatter(out_v, [obase + d], gt)
                gd = plsc.load_gather(ctab_v, [dbase + d])
                plsc.store_scatter(out_v, [obase + EMB + d], gd)
            return 0

        lax.fori_loop(0, GPC, group_body, 0)
        pltpu.sync_copy(out_v, out_hbm.at[pl.ds(cb * OUT_D, C * OUT_D)])
        return 0

    lax.fori_loop(0, NCHUNK, chunk_body, 0)


@jax.jit
def _emb(tod_f, dow_f, ctab):
    mesh = plsc.VectorSubcoreMesh(core_axis_name="c", subcore_axis_name="s")
    f = pl.kernel(
        _body,
        mesh=mesh,
        compiler_params=pltpu.CompilerParams(needs_layout_passes=False),
        out_type=jax.ShapeDtypeStruct((N * OUT_D,), jnp.float32),
        scratch_types=[
            pltpu.VMEM((CTAB_LEN,), jnp.float32),
            pltpu.VMEM((C,), jnp.float32),
            pltpu.VMEM((C,), jnp.int32),
            pltpu.VMEM((C * OUT_D,), jnp.float32),
        ],
    )
    return f(tod_f, dow_f, ctab)


def kernel(tod, dow, tod_table, dow_table):
    ctab = jnp.concatenate([tod_table, dow_table], axis=0).reshape(-1)
    out = _emb(tod.reshape(-1), dow.reshape(-1).astype(jnp.int32), ctab)
    return out.reshape(B, T, OUT_D)


# speedup vs baseline: 6.5485x; 1.2737x over previous
import functools

import jax
import jax.numpy as jnp
from jax import lax
from jax.experimental import pallas as pl
from jax.experimental.pallas import tpu as pltpu
from jax.experimental.pallas import tpu_sc as plsc

B, T = 16384, 200
STEPS_PER_DAY = 288
EMB = 16
OUT_D = 2 * EMB
N = B * T
NW = 32
RPW = N // NW
C = 1280
NCHUNK = RPW // C
GPC = C // 16
DOW_BASE = STEPS_PER_DAY * EMB
CTAB_LEN = DOW_BASE + 7 * EMB


def _body(tod_hbm, dow_hbm, ctab_hbm, out_hbm, ctab_v, tod_v, dow_v, out_v):
    wid = lax.axis_index("s") * 2 + lax.axis_index("c")
    base = wid * RPW
    pltpu.sync_copy(ctab_hbm, ctab_v)
    lane = lax.iota(jnp.int32, 16)

    def chunk_body(ci, _):
        cb = base + ci * C
        pltpu.sync_copy(tod_hbm.at[pl.ds(cb, C)], tod_v)
        pltpu.sync_copy(dow_hbm.at[pl.ds(cb, C)], dow_v)

        @plsc.parallel_loop(0, GPC, 1, unroll=2)
        def group_body(g):
            tv = tod_v[pl.ds(g * 16, 16)]
            dv = dow_v[pl.ds(g * 16, 16)]
            ti = (tv * jnp.float32(STEPS_PER_DAY)).astype(jnp.int32)
            ti = jnp.minimum(jnp.maximum(ti, 0), STEPS_PER_DAY - 1)
            di = jnp.minimum(jnp.maximum(dv, 0), 6)
            tbase = ti * EMB
            dbase = di * EMB + DOW_BASE
            obase = lane * OUT_D + g * (16 * OUT_D)
            for d in range(EMB):
                gt = plsc.load_gather(ctab_v, [tbase + d])
                plsc.store_scatter(out_v, [obase + d], gt)
                gd = plsc.load_gather(ctab_v, [dbase + d])
                plsc.store_scatter(out_v, [obase + EMB + d], gd)
        pltpu.sync_copy(out_v, out_hbm.at[pl.ds(cb * OUT_D, C * OUT_D)])
        return 0

    lax.fori_loop(0, NCHUNK, chunk_body, 0)


@jax.jit
def _emb(tod_f, dow_f, ctab):
    mesh = plsc.VectorSubcoreMesh(core_axis_name="c", subcore_axis_name="s")
    f = pl.kernel(
        _body,
        mesh=mesh,
        compiler_params=pltpu.CompilerParams(needs_layout_passes=False),
        out_type=jax.ShapeDtypeStruct((N * OUT_D,), jnp.float32),
        scratch_types=[
            pltpu.VMEM((CTAB_LEN,), jnp.float32),
            pltpu.VMEM((C,), jnp.float32),
            pltpu.VMEM((C,), jnp.int32),
            pltpu.VMEM((C * OUT_D,), jnp.float32),
        ],
    )
    return f(tod_f, dow_f, ctab)


def kernel(tod, dow, tod_table, dow_table):
    ctab = jnp.concatenate([tod_table, dow_table], axis=0).reshape(-1)
    out = _emb(tod.reshape(-1), dow.reshape(-1).astype(jnp.int32), ctab)
    return out.reshape(B, T, OUT_D)


# speedup vs baseline: 12.2310x; 1.8678x over previous
import functools

import jax
import jax.numpy as jnp
from jax import lax
from jax.experimental import pallas as pl
from jax.experimental.pallas import tpu as pltpu
from jax.experimental.pallas import tpu_sc as plsc

B, T = 16384, 200
STEPS_PER_DAY = 288
EMB = 16
OUT_D = 2 * EMB
N = B * T
NW = 32
RPW = N // NW
C = 1280
NCHUNK = RPW // C
GPC = C // 16
DOW_BASE = STEPS_PER_DAY * EMB
CTAB_LEN = DOW_BASE + 7 * EMB


def _body(tod_hbm, dow_hbm, ctab_hbm, out_hbm, ctab_v, tod_v, dow_v, out_v):
    wid = lax.axis_index("s") * 2 + lax.axis_index("c")
    base = wid * RPW
    pltpu.sync_copy(ctab_hbm, ctab_v)
    lane = lax.iota(jnp.int32, 16)

    def chunk_body(ci, _):
        cb = base + ci * C
        pltpu.sync_copy(tod_hbm.at[pl.ds(cb, C)], tod_v)
        pltpu.sync_copy(dow_hbm.at[pl.ds(cb, C)], dow_v)

        @plsc.parallel_loop(0, GPC, 1, unroll=2)
        def group_body(g):
            tv = tod_v[pl.ds(g * 16, 16)]
            dv = dow_v[pl.ds(g * 16, 16)]
            ti = (tv * jnp.float32(STEPS_PER_DAY)).astype(jnp.int32)
            ti = jnp.minimum(jnp.maximum(ti, 0), STEPS_PER_DAY - 1)
            di = jnp.minimum(jnp.maximum(dv, 0), 6)
            tbase = ti * EMB
            dbase = di * EMB + DOW_BASE
            obase = g * (16 * OUT_D)
            for half in range(2):
                rows = []
                for r in range(8 * half, 8 * half + 8):
                    sel = jnp.full((16,), r, jnp.int32)
                    tb = jnp.take_along_axis(tbase, sel, axis=0) + lane
                    db = jnp.take_along_axis(dbase, sel, axis=0) + lane
                    rows.append(plsc.load_gather(ctab_v, [tb]))
                    rows.append(plsc.load_gather(ctab_v, [db]))
                for i, r in enumerate(range(8 * half, 8 * half + 8)):
                    out_v[pl.ds(obase + r * OUT_D, 16)] = rows[2 * i]
                    out_v[pl.ds(obase + r * OUT_D + EMB, 16)] = rows[2 * i + 1]
        pltpu.sync_copy(out_v, out_hbm.at[pl.ds(cb * OUT_D, C * OUT_D)])
        return 0

    lax.fori_loop(0, NCHUNK, chunk_body, 0)


@jax.jit
def _emb(tod_f, dow_f, ctab):
    mesh = plsc.VectorSubcoreMesh(core_axis_name="c", subcore_axis_name="s")
    f = pl.kernel(
        _body,
        mesh=mesh,
        compiler_params=pltpu.CompilerParams(needs_layout_passes=False),
        out_type=jax.ShapeDtypeStruct((N * OUT_D,), jnp.float32),
        scratch_types=[
            pltpu.VMEM((CTAB_LEN,), jnp.float32),
            pltpu.VMEM((C,), jnp.float32),
            pltpu.VMEM((C,), jnp.int32),
            pltpu.VMEM((C * OUT_D,), jnp.float32),
        ],
    )
    return f(tod_f, dow_f, ctab)


def kernel(tod, dow, tod_table, dow_table):
    ctab = jnp.concatenate([tod_table, dow_table], axis=0).reshape(-1)
    out = _emb(tod.reshape(-1), dow.reshape(-1).astype(jnp.int32), ctab)
    return out.reshape(B, T, OUT_D)


# speedup vs baseline: 13.7612x; 1.1251x over previous
import jax
import jax.numpy as jnp
from jax import lax
from jax.experimental import pallas as pl
from jax.experimental.pallas import tpu as pltpu
from jax.experimental.pallas import tpu_sc as plsc

B, T = 16384, 200
STEPS_PER_DAY = 288
EMB = 16
OUT_D = 2 * EMB
N = B * T
NW = 32
RPW = N // NW
C = 1280
NCHUNK = RPW // C
GPC = C // 16
DOW_BASE = STEPS_PER_DAY * EMB
CTAB_LEN = DOW_BASE + 7 * EMB


def _body(tod_hbm, dow_hbm, ctab_hbm, out_hbm, ctab_v,
          tod_v0, tod_v1, dow_v0, dow_v1, out_v0, out_v1,
          st0, st1, sd0, sd1, so0, so1):
    wid = lax.axis_index("s") * 2 + lax.axis_index("c")
    base = wid * RPW
    pltpu.sync_copy(ctab_hbm, ctab_v)
    lane = lax.iota(jnp.int32, 16)
    tod_b = (tod_v0, tod_v1)
    dow_b = (dow_v0, dow_v1)
    out_b = (out_v0, out_v1)
    sin_t = (st0, st1)
    sin_d = (sd0, sd1)
    sout = (so0, so1)

    def start_in(ci, b):
        cb = base + ci * C
        pltpu.async_copy(tod_hbm.at[pl.ds(cb, C)], tod_b[b], sin_t[b])
        pltpu.async_copy(dow_hbm.at[pl.ds(cb, C)], dow_b[b], sin_d[b])

    def wait_in(b):
        pltpu.make_async_copy(tod_hbm.at[pl.ds(0, C)], tod_b[b], sin_t[b]).wait()
        pltpu.make_async_copy(dow_hbm.at[pl.ds(0, C)], dow_b[b], sin_d[b]).wait()

    def wait_out(b):
        pltpu.make_async_copy(
            out_b[b], out_hbm.at[pl.ds(0, C * OUT_D)], sout[b]).wait()

    def compute_chunk(b):
        tod_v, dow_v, out_v = tod_b[b], dow_b[b], out_b[b]

        @plsc.parallel_loop(0, GPC, 1, unroll=2)
        def group_body(g):
            tv = tod_v[pl.ds(g * 16, 16)]
            dv = dow_v[pl.ds(g * 16, 16)]
            ti = (tv * jnp.float32(STEPS_PER_DAY)).astype(jnp.int32)
            ti = jnp.minimum(jnp.maximum(ti, 0), STEPS_PER_DAY - 1)
            di = jnp.minimum(jnp.maximum(dv, 0), 6)
            tbase = ti * EMB
            dbase = di * EMB + DOW_BASE
            obase = g * (16 * OUT_D)
            for half in range(2):
                rows = []
                for r in range(8 * half, 8 * half + 8):
                    sel = jnp.full((16,), r, jnp.int32)
                    tb = jnp.take_along_axis(tbase, sel, axis=0) + lane
                    db = jnp.take_along_axis(dbase, sel, axis=0) + lane
                    rows.append(plsc.load_gather(ctab_v, [tb]))
                    rows.append(plsc.load_gather(ctab_v, [db]))
                for i, r in enumerate(range(8 * half, 8 * half + 8)):
                    out_v[pl.ds(obase + r * OUT_D, 16)] = rows[2 * i]
                    out_v[pl.ds(obase + r * OUT_D + EMB, 16)] = rows[2 * i + 1]

    start_in(0, 0)
    start_in(1, 1)

    def chunk_pair(k, _):
        for b in range(2):
            ci = 2 * k + b
            wait_in(b)

            @pl.when(k >= 1)
            def _():
                wait_out(b)

            compute_chunk(b)
            cb = base + ci * C
            pltpu.async_copy(
                out_b[b], out_hbm.at[pl.ds(cb * OUT_D, C * OUT_D)], sout[b])

            @pl.when(k < NCHUNK // 2 - 1)
            def _():
                start_in(ci + 2, b)

        return 0

    lax.fori_loop(0, NCHUNK // 2, chunk_pair, 0)
    wait_out(0)
    wait_out(1)


@jax.jit
def _emb(tod_f, dow_f, ctab):
    mesh = plsc.VectorSubcoreMesh(core_axis_name="c", subcore_axis_name="s")
    f = pl.kernel(
        _body,
        mesh=mesh,
        compiler_params=pltpu.CompilerParams(needs_layout_passes=False),
        out_type=jax.ShapeDtypeStruct((N * OUT_D,), jnp.float32),
        scratch_types=[
            pltpu.VMEM((CTAB_LEN,), jnp.float32),
            pltpu.VMEM((C,), jnp.float32),
            pltpu.VMEM((C,), jnp.float32),
            pltpu.VMEM((C,), jnp.int32),
            pltpu.VMEM((C,), jnp.int32),
            pltpu.VMEM((C * OUT_D,), jnp.float32),
            pltpu.VMEM((C * OUT_D,), jnp.float32),
            pltpu.SemaphoreType.DMA,
            pltpu.SemaphoreType.DMA,
            pltpu.SemaphoreType.DMA,
            pltpu.SemaphoreType.DMA,
            pltpu.SemaphoreType.DMA,
            pltpu.SemaphoreType.DMA,
        ],
    )
    return f(tod_f, dow_f, ctab)


def kernel(tod, dow, tod_table, dow_table):
    ctab = jnp.concatenate([tod_table, dow_table], axis=0).reshape(-1)
    out = _emb(tod.reshape(-1), dow.reshape(-1).astype(jnp.int32), ctab)
    return out.reshape(B, T, OUT_D)


# speedup vs baseline: 82.9639x; 6.0288x over previous
import jax
import jax.numpy as jnp
from jax import lax
from jax.experimental import pallas as pl
from jax.experimental.pallas import tpu as pltpu
from jax.experimental.pallas import tpu_sc as plsc

B, T = 16384, 200
STEPS_PER_DAY = 288
EMB = 16
OUT_D = 2 * EMB
N = B * T
NW = 32
CB = 1024
NCHUNK = N // CB
CPW = NCHUNK // NW
GPC = CB // 16
DOW_BASE = EMB * STEPS_PER_DAY
DOW_D = 7 * 16
CTAB_LEN = DOW_BASE + EMB * DOW_D


def _body(tod_hbm, dow_hbm, ctab_hbm, out_hbm, ctab_v,
          tod_v0, tod_v1, dow_v0, dow_v1, out_v0, out_v1,
          st0, st1, sd0, sd1, so0, so1):
    wid = lax.axis_index("s") * 2 + lax.axis_index("c")
    cid0 = wid * CPW
    pltpu.sync_copy(ctab_hbm, ctab_v)
    lane = lax.iota(jnp.int32, 16)
    tod_b = (tod_v0, tod_v1)
    dow_b = (dow_v0, dow_v1)
    out_b = (out_v0, out_v1)
    sin_t = (st0, st1)
    sin_d = (sd0, sd1)
    sout = (so0, so1)

    def start_in(cid, b):
        pltpu.async_copy(tod_hbm.at[pl.ds(cid * CB, CB)], tod_b[b], sin_t[b])
        pltpu.async_copy(dow_hbm.at[pl.ds(cid * CB, CB)], dow_b[b], sin_d[b])

    def wait_in(b):
        pltpu.make_async_copy(tod_hbm.at[pl.ds(0, CB)], tod_b[b], sin_t[b]).wait()
        pltpu.make_async_copy(dow_hbm.at[pl.ds(0, CB)], dow_b[b], sin_d[b]).wait()

    def start_out(cid, b):
        t = cid >> 4
        bq = cid & 15
        for dt in range(4):
            pltpu.async_copy(
                out_b[b].at[dt], out_hbm.at[t, dt, pl.ds(bq * 8, 8)], sout[b])

    def wait_out(b):
        for dt in range(4):
            pltpu.make_async_copy(
                out_b[b].at[dt], out_hbm.at[0, 0, pl.ds(0, 8)], sout[b]).wait()

    def compute_chunk(b):
        tod_v, dow_v, out_v = tod_b[b], dow_b[b], out_b[b]

        @plsc.parallel_loop(0, GPC, 1, unroll=2)
        def group_body(g):
            btl = g >> 3
            j16 = (g & 7) * 16
            tv = tod_v[pl.ds(g * 16, 16)]
            dv = dow_v[pl.ds(g * 16, 16)]
            ti = (tv * jnp.float32(STEPS_PER_DAY)).astype(jnp.int32)
            ti = jnp.minimum(jnp.maximum(ti, 0), STEPS_PER_DAY - 1)
            di = jnp.minimum(jnp.maximum(dv, 0), 6)
            dbase = di * 16 + lane + DOW_BASE
            tvals = [plsc.load_gather(ctab_v, [ti + d * STEPS_PER_DAY])
                     for d in range(EMB)]
            dvals = [plsc.load_gather(ctab_v, [dbase + d * DOW_D])
                     for d in range(EMB)]
            for d in range(OUT_D):
                val = tvals[d] if d < EMB else dvals[d - EMB]
                out_v[d // 8, btl, d % 8, pl.ds(j16, 16)] = val

    start_in(cid0, 0)
    start_in(cid0 + 1, 1)

    def chunk_pair(k, _):
        for b in range(2):
            ci = cid0 + 2 * k + b
            wait_in(b)

            @pl.when(k >= 1)
            def _():
                wait_out(b)

            compute_chunk(b)
            start_out(ci, b)

            @pl.when(k < CPW // 2 - 1)
            def _():
                start_in(ci + 2, b)

        return 0

    lax.fori_loop(0, CPW // 2, chunk_pair, 0)
    wait_out(0)
    wait_out(1)


@jax.jit
def _emb(tod_f, dow_f, ctab):
    mesh = plsc.VectorSubcoreMesh(core_axis_name="c", subcore_axis_name="s")
    f = pl.kernel(
        _body,
        mesh=mesh,
        compiler_params=pltpu.CompilerParams(needs_layout_passes=False),
        out_type=jax.ShapeDtypeStruct((T, 4, 128, 8, 128), jnp.float32),
        scratch_types=[
            pltpu.VMEM((CTAB_LEN,), jnp.float32),
            pltpu.VMEM((CB,), jnp.float32),
            pltpu.VMEM((CB,), jnp.float32),
            pltpu.VMEM((CB,), jnp.int32),
            pltpu.VMEM((CB,), jnp.int32),
            pltpu.VMEM((4, 8, 8, 128), jnp.float32),
            pltpu.VMEM((4, 8, 8, 128), jnp.float32),
            pltpu.SemaphoreType.DMA,
            pltpu.SemaphoreType.DMA,
            pltpu.SemaphoreType.DMA,
            pltpu.SemaphoreType.DMA,
            pltpu.SemaphoreType.DMA,
            pltpu.SemaphoreType.DMA,
        ],
    )
    return f(tod_f, dow_f, ctab)


def kernel(tod, dow, tod_table, dow_table):
    tpart = tod_table.T.reshape(-1)
    dpart = jnp.broadcast_to(
        dow_table.T[:, :, None], (EMB, 7, 16)).reshape(-1)
    ctab = jnp.concatenate([tpart, dpart])
    tod_f = tod.T.reshape(-1)
    dow_f = dow.T.reshape(-1).astype(jnp.int32)
    out = _emb(tod_f, dow_f, ctab)
    return out.transpose(2, 4, 0, 1, 3).reshape(B, T, OUT_D)
